# baseline (speedup 1.0000x reference)
import functools

import jax
import jax.numpy as jnp
from jax import lax
from jax.experimental import pallas as pl
from jax.experimental.pallas import tpu as pltpu
from jax.experimental.pallas import tpu_sc as plsc

N_NODES = 10000
N_EDGES = 320000
D = 128

NC = 2
NS = 16
NW = NC * NS
FPW = D // NW
CE = 6400
NCHUNK = N_EDGES // CE
GROUPS = CE // 16
L = 16


U = 4


def _make_segsum(with_count):
    mesh = plsc.VectorSubcoreMesh(core_axis_name="c", subcore_axis_name="s")
    out_type = [jax.ShapeDtypeStruct((D * N_NODES,), jnp.float32)]
    SLICE = FPW * N_NODES
    scratch = [
        pltpu.VMEM((SLICE,), jnp.float32),
        pltpu.VMEM((SLICE,), jnp.float32),
        pltpu.VMEM((CE,), jnp.int32),
        pltpu.VMEM((CE,), jnp.int32),
        pltpu.VMEM((CE,), jnp.int32),
        pltpu.VMEM((CE,), jnp.int32),
        pltpu.SemaphoreType.DMA,
        pltpu.SemaphoreType.DMA,
        pltpu.SemaphoreType.DMA,
        pltpu.SemaphoreType.DMA,
    ]
    if with_count:
        out_type.append(jax.ShapeDtypeStruct((NW, N_NODES), jnp.float32))
        scratch.append(pltpu.VMEM((N_NODES,), jnp.float32))
        scratch.append(pltpu.VMEM((N_EDGES // NW,), jnp.int32))

    def body(xT, src, dst, *refs):
        if with_count:
            (sum_out, cnt_out, tab, acc, srcb0, dstb0, srcb1, dstb1,
             ss0, sd0, ss1, sd1, cntb, idxb) = refs
        else:
            (sum_out, tab, acc, srcb0, dstb0, srcb1, dstb1,
             ss0, sd0, ss1, sd1) = refs
        bufs = ((srcb0, dstb0, ss0, sd0), (srcb1, dstb1, ss1, sd1))
        wid = lax.axis_index("s") * NC + lax.axis_index("c")

        CP = CE

        def start(e, sb, db, ssem, dsem):
            pltpu.make_async_copy(src.at[pl.ds(e * CP, CP)], sb, ssem).start()
            pltpu.make_async_copy(dst.at[pl.ds(e * CP, CP)], db, dsem).start()

        def wait(sb, db, ssem, dsem):
            pltpu.make_async_copy(src.at[pl.ds(0, CP)], sb, ssem).wait()
            pltpu.make_async_copy(dst.at[pl.ds(0, CP)], db, dsem).wait()

        start(0, *bufs[0])
        start(1, *bufs[1])

        pltpu.sync_copy(xT.at[pl.ds(wid * SLICE, SLICE)], tab)

        zero = jnp.zeros((L,), jnp.float32)

        ZU = 5

        def zbody(i, carry):
            for j in range(ZU):
                acc[pl.ds(i * (L * ZU) + j * L, L)] = zero
            return carry

        lax.fori_loop(0, SLICE // (L * ZU), zbody, 0)

        if with_count:
            def zc(i, carry):
                for j in range(ZU):
                    cntb[pl.ds(i * (L * ZU) + j * L, L)] = zero
                return carry
            lax.fori_loop(0, N_NODES // (L * ZU), zc, 0)

        ones16 = jnp.full((L,), 1.0, jnp.float32)

        def process(sb, db):
            def grp(g):
                base = g * (L * U)
                svecs = [sb[pl.ds(base + u * L, L)] for u in range(U)]
                dvecs = [db[pl.ds(base + u * L, L)] for u in range(U)]
                vals = [plsc.load_gather(tab, [svecs[u] + (c * N_NODES)])
                        for u in range(U) for c in range(FPW)]
                i = 0
                for u in range(U):
                    for c in range(FPW):
                        plsc.addupdate_scatter(
                            acc, [dvecs[u] + (c * N_NODES)], vals[i])
                        i += 1
            plsc.parallel_loop(0, GROUPS // U)(grp)

        def chunk_body(k, carry):
            for b in range(2):
                e = k * 2 + b
                sb, db, ssem, dsem = bufs[b]
                wait(sb, db, ssem, dsem)
                process(sb, db)

                @pl.when(e + 2 < NCHUNK)
                def _():
                    start(e + 2, sb, db, ssem, dsem)
            return carry

        lax.fori_loop(0, NCHUNK // 2, chunk_body, 0)

        pltpu.sync_copy(acc, sum_out.at[pl.ds(wid * SLICE, SLICE)])
        if with_count:
            SHARD = N_EDGES // NW
            CU = 5
            pltpu.sync_copy(dst.at[pl.ds(wid * SHARD, SHARD)], idxb)

            @plsc.parallel_loop(0, SHARD // (L * CU))
            def _(g):
                for j in range(CU):
                    dv = idxb[pl.ds(g * (L * CU) + j * L, L)]
                    plsc.addupdate_scatter(cntb, [dv], ones16)

            pltpu.sync_copy(cntb, cnt_out.at[wid])

    return functools.partial(
        pl.kernel, mesh=mesh, out_type=out_type, scratch_types=scratch,
        compiler_params=pltpu.CompilerParams(needs_layout_passes=False),
    )(body)


NB = 1024
GRID = (N_NODES + NB - 1) // NB


def _transpose(x):
    def tbody(x_ref, o_ref):
        o_ref[...] = x_ref[...].T

    return pl.pallas_call(
        tbody,
        grid=(GRID,),
        in_specs=[pl.BlockSpec((NB, D), lambda i: (i, 0))],
        out_specs=pl.BlockSpec((D, NB), lambda i: (0, i)),
        out_shape=jax.ShapeDtypeStruct((D, N_NODES), jnp.float32),
    )(x)


def _layer(sumT, cnt, hT, Wl, Wr, b, relu, transpose_out):
    cnt_rows = cnt.shape[0]

    def lbody(s_ref, c_ref, h_ref, wl_ref, wr_ref, b_ref, *o_refs):
        if cnt_rows > 1:
            inv = 1.0 / jnp.maximum(
                jnp.sum(c_ref[...], axis=0, keepdims=True), 1.0)
        else:
            inv = c_ref[...]
        mean = s_ref[...] * inv
        acc = jnp.dot(wl_ref[...], mean, preferred_element_type=jnp.float32)
        acc = acc + jnp.dot(wr_ref[...], h_ref[...],
                            preferred_element_type=jnp.float32)
        acc = acc + b_ref[...]
        if relu:
            acc = jnp.maximum(acc, 0.0)
        o_refs[0][...] = acc.T if transpose_out else acc
        if cnt_rows > 1:
            o_refs[1][...] = inv

    out_shape = [jax.ShapeDtypeStruct(
        (N_NODES, D) if transpose_out else (D, N_NODES), jnp.float32)]
    out_specs = [pl.BlockSpec((NB, D), lambda i: (i, 0)) if transpose_out
                 else pl.BlockSpec((D, NB), lambda i: (0, i))]
    if cnt_rows > 1:
        out_shape.append(jax.ShapeDtypeStruct((1, N_NODES), jnp.float32))
        out_specs.append(pl.BlockSpec((1, NB), lambda i: (0, i)))
    return pl.pallas_call(
        lbody,
        grid=(GRID,),
        in_specs=[
            pl.BlockSpec((D, NB), lambda i: (0, i)),
            pl.BlockSpec((cnt_rows, NB), lambda i: (0, i)),
            pl.BlockSpec((D, NB), lambda i: (0, i)),
            pl.BlockSpec((D, D), lambda i: (0, 0)),
            pl.BlockSpec((D, D), lambda i: (0, 0)),
            pl.BlockSpec((D, 1), lambda i: (0, 0)),
        ],
        out_specs=out_specs,
        out_shape=out_shape,
    )(sumT, cnt, hT, Wl, Wr, b)


_segsum_with_cnt = _make_segsum(True)
_segsum = _make_segsum(False)


def kernel(x, edge_index, W1_l, W1_r, b1, W2_l, W2_r, b2):
    src = edge_index[0].astype(jnp.int32)
    dst = edge_index[1].astype(jnp.int32)
    xT = _transpose(x)
    sum1T, cnt = _segsum_with_cnt(xT.reshape(-1), src, dst)
    sum1T = sum1T.reshape(D, N_NODES)
    hT, inv = _layer(sum1T, cnt, xT, W1_l, W1_r, b1.reshape(D, 1),
                     relu=True, transpose_out=False)
    (sum2T,) = _segsum(hT.reshape(-1), src, dst)
    sum2T = sum2T.reshape(D, N_NODES)
    out, = _layer(sum2T, inv, hT, W2_l, W2_r, b2.reshape(D, 1),
                  relu=False, transpose_out=True)
    return out


# speedup vs baseline: 1.0059x; 1.0059x over previous
import functools

import jax
import jax.numpy as jnp
from jax import lax
from jax.experimental import pallas as pl
from jax.experimental.pallas import tpu as pltpu
from jax.experimental.pallas import tpu_sc as plsc

N_NODES = 10000
N_EDGES = 320000
D = 128

NC = 2
NS = 16
NW = NC * NS
FPW = D // NW
CE = 6400
NCHUNK = N_EDGES // CE
GROUPS = CE // 16
L = 16


U = 4


def _make_segsum(with_count):
    mesh = plsc.VectorSubcoreMesh(core_axis_name="c", subcore_axis_name="s")
    out_type = [jax.ShapeDtypeStruct((D * N_NODES,), jnp.float32)]
    SLICE = FPW * N_NODES
    scratch = [
        pltpu.VMEM((SLICE,), jnp.float32),
        pltpu.VMEM((SLICE,), jnp.float32),
        pltpu.VMEM((CE,), jnp.int32),
        pltpu.VMEM((CE,), jnp.int32),
        pltpu.VMEM((CE,), jnp.int32),
        pltpu.VMEM((CE,), jnp.int32),
        pltpu.SemaphoreType.DMA,
        pltpu.SemaphoreType.DMA,
        pltpu.SemaphoreType.DMA,
        pltpu.SemaphoreType.DMA,
    ]
    if with_count:
        out_type.append(jax.ShapeDtypeStruct((NW, N_NODES), jnp.float32))
        scratch.append(pltpu.VMEM((N_NODES,), jnp.float32))
        scratch.append(pltpu.VMEM((N_EDGES // NW,), jnp.int32))

    def body(xT, src, dst, *refs):
        if with_count:
            (sum_out, cnt_out, tab, acc, srcb0, dstb0, srcb1, dstb1,
             ss0, sd0, ss1, sd1, cntb, idxb) = refs
        else:
            (sum_out, tab, acc, srcb0, dstb0, srcb1, dstb1,
             ss0, sd0, ss1, sd1) = refs
        bufs = ((srcb0, dstb0, ss0, sd0), (srcb1, dstb1, ss1, sd1))
        wid = lax.axis_index("s") * NC + lax.axis_index("c")

        CP = CE

        def start(e, sb, db, ssem, dsem):
            pltpu.make_async_copy(src.at[pl.ds(e * CP, CP)], sb, ssem).start()
            pltpu.make_async_copy(dst.at[pl.ds(e * CP, CP)], db, dsem).start()

        def wait(sb, db, ssem, dsem):
            pltpu.make_async_copy(src.at[pl.ds(0, CP)], sb, ssem).wait()
            pltpu.make_async_copy(dst.at[pl.ds(0, CP)], db, dsem).wait()

        pltpu.sync_copy(xT.at[pl.ds(wid * SLICE, SLICE)], tab)

        zero = jnp.zeros((L,), jnp.float32)

        ZU = 5

        def zbody(i, carry):
            for j in range(ZU):
                acc[pl.ds(i * (L * ZU) + j * L, L)] = zero
            return carry

        lax.fori_loop(0, SLICE // (L * ZU), zbody, 0)

        if with_count:
            def zc(i, carry):
                for j in range(ZU):
                    cntb[pl.ds(i * (L * ZU) + j * L, L)] = zero
                return carry
            lax.fori_loop(0, N_NODES // (L * ZU), zc, 0)

        ones16 = jnp.full((L,), 1.0, jnp.float32)

        def process(sb, db):
            def grp(g):
                base = g * (L * U)
                svecs = [sb[pl.ds(base + u * L, L)] for u in range(U)]
                dvecs = [db[pl.ds(base + u * L, L)] for u in range(U)]
                vals = [plsc.load_gather(tab, [svecs[u] + (c * N_NODES)])
                        for u in range(U) for c in range(FPW)]
                i = 0
                for u in range(U):
                    for c in range(FPW):
                        plsc.addupdate_scatter(
                            acc, [dvecs[u] + (c * N_NODES)], vals[i])
                        i += 1
            plsc.parallel_loop(0, GROUPS // U)(grp)

        start(0, *bufs[0])
        start(1, *bufs[1])

        def chunk_body(k, carry):
            for b in range(2):
                e = k * 2 + b
                sb, db, ssem, dsem = bufs[b]
                wait(sb, db, ssem, dsem)
                process(sb, db)

                @pl.when(e + 2 < NCHUNK)
                def _():
                    start(e + 2, sb, db, ssem, dsem)
            return carry

        lax.fori_loop(0, NCHUNK // 2, chunk_body, 0)

        pltpu.sync_copy(acc, sum_out.at[pl.ds(wid * SLICE, SLICE)])
        if with_count:
            SHARD = N_EDGES // NW
            CU = 5
            pltpu.sync_copy(dst.at[pl.ds(wid * SHARD, SHARD)], idxb)

            @plsc.parallel_loop(0, SHARD // (L * CU))
            def _(g):
                for j in range(CU):
                    dv = idxb[pl.ds(g * (L * CU) + j * L, L)]
                    plsc.addupdate_scatter(cntb, [dv], ones16)

            pltpu.sync_copy(cntb, cnt_out.at[wid])

    return functools.partial(
        pl.kernel, mesh=mesh, out_type=out_type, scratch_types=scratch,
        compiler_params=pltpu.CompilerParams(needs_layout_passes=False),
    )(body)


NB = 1024
GRID = (N_NODES + NB - 1) // NB


def _transpose(x):
    def tbody(x_ref, o_ref):
        o_ref[...] = x_ref[...].T

    return pl.pallas_call(
        tbody,
        grid=(GRID,),
        in_specs=[pl.BlockSpec((NB, D), lambda i: (i, 0))],
        out_specs=pl.BlockSpec((D, NB), lambda i: (0, i)),
        out_shape=jax.ShapeDtypeStruct((D, N_NODES), jnp.float32),
    )(x)


def _layer(sumT, cnt, hT, Wl, Wr, b, relu, transpose_out):
    cnt_rows = cnt.shape[0]

    def lbody(s_ref, c_ref, h_ref, wl_ref, wr_ref, b_ref, *o_refs):
        if cnt_rows > 1:
            inv = 1.0 / jnp.maximum(
                jnp.sum(c_ref[...], axis=0, keepdims=True), 1.0)
        else:
            inv = c_ref[...]
        mean = s_ref[...] * inv
        acc = jnp.dot(wl_ref[...], mean, preferred_element_type=jnp.float32)
        acc = acc + jnp.dot(wr_ref[...], h_ref[...],
                            preferred_element_type=jnp.float32)
        acc = acc + b_ref[...]
        if relu:
            acc = jnp.maximum(acc, 0.0)
        o_refs[0][...] = acc.T if transpose_out else acc
        if cnt_rows > 1:
            o_refs[1][...] = inv

    out_shape = [jax.ShapeDtypeStruct(
        (N_NODES, D) if transpose_out else (D, N_NODES), jnp.float32)]
    out_specs = [pl.BlockSpec((NB, D), lambda i: (i, 0)) if transpose_out
                 else pl.BlockSpec((D, NB), lambda i: (0, i))]
    if cnt_rows > 1:
        out_shape.append(jax.ShapeDtypeStruct((1, N_NODES), jnp.float32))
        out_specs.append(pl.BlockSpec((1, NB), lambda i: (0, i)))
    return pl.pallas_call(
        lbody,
        grid=(GRID,),
        in_specs=[
            pl.BlockSpec((D, NB), lambda i: (0, i)),
            pl.BlockSpec((cnt_rows, NB), lambda i: (0, i)),
            pl.BlockSpec((D, NB), lambda i: (0, i)),
            pl.BlockSpec((D, D), lambda i: (0, 0)),
            pl.BlockSpec((D, D), lambda i: (0, 0)),
            pl.BlockSpec((D, 1), lambda i: (0, 0)),
        ],
        out_specs=out_specs,
        out_shape=out_shape,
    )(sumT, cnt, hT, Wl, Wr, b)


_segsum_with_cnt = _make_segsum(True)
_segsum = _make_segsum(False)


def kernel(x, edge_index, W1_l, W1_r, b1, W2_l, W2_r, b2):
    src = edge_index[0].astype(jnp.int32)
    dst = edge_index[1].astype(jnp.int32)
    xT = _transpose(x)
    sum1T, cnt = _segsum_with_cnt(xT.reshape(-1), src, dst)
    sum1T = sum1T.reshape(D, N_NODES)
    hT, inv = _layer(sum1T, cnt, xT, W1_l, W1_r, b1.reshape(D, 1),
                     relu=True, transpose_out=False)
    (sum2T,) = _segsum(hT.reshape(-1), src, dst)
    sum2T = sum2T.reshape(D, N_NODES)
    out, = _layer(sum2T, inv, hT, W2_l, W2_r, b2.reshape(D, 1),
                  relu=False, transpose_out=True)
    return out
